# baseline (speedup 1.0000x reference)
import functools

import jax
import jax.numpy as jnp
from jax import lax
from jax.experimental import pallas as pl
from jax.experimental.pallas import tpu as pltpu
from jax.experimental.pallas import tpu_sc as plsc

_NC = 2
_NS = 16
_NW = _NC * _NS
_CH = 2
_F = 256
_BB = 128


def _sc_body(nchunk, S, D, x_hbm, tbl_hbm, out_hbm,
             tblv, buf0, buf1, tsem, isem0, isem1, osem0, osem1):
    wid = lax.axis_index("s") * _NC + lax.axis_index("c")
    base = wid * (nchunk * _CH)
    bufs = (buf0, buf1)
    isems = (isem0, isem1)
    osems = (osem0, osem1)

    tcp = pltpu.make_async_copy(tbl_hbm, tblv, tsem)
    tcp.start()
    tcp.wait()

    def in_copy(k):
        return pltpu.make_async_copy(
            x_hbm.at[pl.ds(base + k * _CH, _CH)], bufs[k % 2], isems[k % 2]
        )

    def out_copy(k):
        return pltpu.make_async_copy(
            bufs[k % 2], out_hbm.at[pl.ds(base + k * _CH, _CH)], osems[k % 2]
        )

    in_copy(0).start()
    for k in range(nchunk):
        in_copy(k).wait()
        b = bufs[k % 2]

        @plsc.parallel_loop(0, S, 1, unroll=2)
        def _add(s):
            for jj in range(D // 16):
                t = tblv[s, pl.ds(jj * 16, 16)]
                for c in range(_CH):
                    b[c, s, pl.ds(jj * 16, 16)] = b[c, s, pl.ds(jj * 16, 16)] + t

        out_copy(k).start()
        if k + 1 < nchunk:
            if k >= 1:
                out_copy(k - 1).wait()
            in_copy(k + 1).start()
    for k in range(max(0, nchunk - 2), nchunk):
        out_copy(k).wait()


def _sc_part(x_part, tbl):
    Bp, S, D = x_part.shape
    nchunk = Bp // (_NW * _CH)
    mesh = plsc.VectorSubcoreMesh(core_axis_name="c", subcore_axis_name="s")
    body = functools.partial(_sc_body, nchunk, S, D)
    return pl.kernel(
        body,
        out_type=jax.ShapeDtypeStruct((Bp, S, D), jnp.float32),
        mesh=mesh,
        scratch_types=[
            pltpu.VMEM((S, D), jnp.float32),
            pltpu.VMEM((_CH, S, D), jnp.float32),
            pltpu.VMEM((_CH, S, D), jnp.float32),
            pltpu.SemaphoreType.DMA,
            pltpu.SemaphoreType.DMA,
            pltpu.SemaphoreType.DMA,
            pltpu.SemaphoreType.DMA,
            pltpu.SemaphoreType.DMA,
        ],
    )(x_part, tbl)


def _tc_add(x_ref, w_ref, o_ref):
    o_ref[...] = x_ref[...] + w_ref[...][None, :, :]


def _tc_part(x_part, tbl):
    Bp, S, D = x_part.shape
    return pl.pallas_call(
        _tc_add,
        grid=(Bp // _BB,),
        in_specs=[
            pl.BlockSpec((_BB, S, D), lambda i: (i, 0, 0)),
            pl.BlockSpec((S, D), lambda i: (0, 0)),
        ],
        out_specs=pl.BlockSpec((_BB, S, D), lambda i: (i, 0, 0)),
        out_shape=jax.ShapeDtypeStruct((Bp, S, D), x_part.dtype),
    )(x_part, tbl)


def kernel(x, pos_emb_weight):
    B, S, D = x.shape
    tbl = pos_emb_weight[:S]
    y_sc = _sc_part(x[:_F], tbl)
    y_tc = _tc_part(x[_F:], tbl)
    return jnp.concatenate([y_sc, y_tc], axis=0)


# speedup vs baseline: 3.3314x; 3.3314x over previous
import jax
import jax.numpy as jnp
from jax.experimental import pallas as pl

_BB = 128


def _add_kernel(x_ref, w_ref, o_ref):
    o_ref[...] = x_ref[...] + w_ref[...][None, :, :]


def kernel(x, pos_emb_weight):
    B, S, D = x.shape
    table = pos_emb_weight[:S]
    grid = (B // _BB,)
    return pl.pallas_call(
        _add_kernel,
        grid=grid,
        in_specs=[
            pl.BlockSpec((_BB, S, D), lambda i: (i, 0, 0)),
            pl.BlockSpec((S, D), lambda i: (0, 0)),
        ],
        out_specs=pl.BlockSpec((_BB, S, D), lambda i: (i, 0, 0)),
        out_shape=jax.ShapeDtypeStruct((B, S, D), x.dtype),
    )(x, table)
